# Initial kernel scaffold; baseline (speedup 1.0000x reference)
import jax
import jax.numpy as jnp
from jax.experimental import pallas as pl


def kernel(hidden_states, W_A, W_gate, W_up, W_down, gate_scale, gate_bias):
    raise NotImplementedError("write your pallas kernel here")


# speedup vs baseline: 4.3468x; 4.3468x over previous
import jax
import jax.numpy as jnp
from jax.experimental import pallas as pl
from jax.experimental.pallas import tpu as pltpu

E = 8
R = 8
D = 768
DFF = 128
GATE_THRESHOLD = 0.5
GATE_TEMPERATURE = 1.0

_T = 512


def _moe_block_kernel(x_ref, wa_ref, wg_ref, wu_ref, wd_ref, sb_ref,
                      out_ref, gs_ref):
    x = x_ref[...]

    gh = jax.lax.dot_general(x.astype(jnp.bfloat16),
                             wa_ref[...].astype(jnp.bfloat16),
                             (((1,), (1,)), ((), ())),
                             preferred_element_type=jnp.float32)
    gh2 = gh * gh
    row = jax.lax.broadcasted_iota(jnp.int32, (E * R, E), 0)
    col = jax.lax.broadcasted_iota(jnp.int32, (E * R, E), 1)
    group = (row // R == col).astype(jnp.float32)
    gh2_hi = gh2.astype(jnp.bfloat16).astype(jnp.float32)
    gh2_lo = gh2 - gh2_hi
    dot = lambda a, b: jax.lax.dot_general(
        a, b, (((1,), (0,)), ((), ())), preferred_element_type=jnp.float32)
    ss = dot(gh2_hi, group) + dot(gh2_lo, group)
    scores = jnp.sqrt(ss * (1.0 / R) + 1e-6)

    threshold = GATE_THRESHOLD / GATE_TEMPERATURE
    s = scores * sb_ref[0:1, :] - sb_ref[1:2, :]
    m = s >= threshold
    sm = jnp.where(m, s, 0.0)
    gs_ref[...] = jnp.where(m, s, -jnp.inf)

    hg = jax.lax.dot_general(x, wg_ref[...], (((1,), (1,)), ((), ())),
                             preferred_element_type=jnp.float32)
    hu = jax.lax.dot_general(x, wu_ref[...], (((1,), (1,)), ((), ())),
                             preferred_element_type=jnp.float32)
    h = (hg * jax.lax.logistic(hg)) * hu

    erow = jax.lax.broadcasted_iota(jnp.int32, (E, E * DFF), 0)
    ecol = jax.lax.broadcasted_iota(jnp.int32, (E, E * DFF), 1)
    expand = (ecol // DFF == erow).astype(jnp.float32)
    sm_hi = sm.astype(jnp.bfloat16).astype(jnp.float32)
    sm_lo = sm - sm_hi
    dot2 = lambda a, b: jax.lax.dot_general(
        a, b, (((1,), (0,)), ((), ())), preferred_element_type=jnp.float32)
    sm_big = dot2(sm_hi, expand) + dot2(sm_lo, expand)
    hs = h * sm_big

    out_ref[...] = jax.lax.dot_general(hs, wd_ref[...],
                                       (((1,), (0,)), ((), ())),
                                       preferred_element_type=jnp.float32)


def kernel(hidden_states, W_A, W_gate, W_up, W_down, gate_scale, gate_bias):
    orig_shape = hidden_states.shape
    x = hidden_states.reshape(-1, orig_shape[-1])
    N = x.shape[0]

    wa = W_A.reshape(E * R, D)
    wg = W_gate.reshape(E * DFF, D)
    wu = W_up.reshape(E * DFF, D)
    wd = jnp.transpose(W_down, (0, 2, 1)).reshape(E * DFF, D)
    sb = jnp.stack([gate_scale, gate_bias], axis=0)

    grid = (N // _T,)
    out, gs = pl.pallas_call(
        _moe_block_kernel,
        grid=grid,
        in_specs=[
            pl.BlockSpec((_T, D), lambda i: (i, 0)),
            pl.BlockSpec((E * R, D), lambda i: (0, 0)),
            pl.BlockSpec((E * DFF, D), lambda i: (0, 0)),
            pl.BlockSpec((E * DFF, D), lambda i: (0, 0)),
            pl.BlockSpec((E * DFF, D), lambda i: (0, 0)),
            pl.BlockSpec((2, E), lambda i: (0, 0)),
        ],
        out_specs=[
            pl.BlockSpec((_T, D), lambda i: (i, 0)),
            pl.BlockSpec((_T, E), lambda i: (i, 0)),
        ],
        out_shape=[
            jax.ShapeDtypeStruct((N, D), jnp.float32),
            jax.ShapeDtypeStruct((N, E), jnp.float32),
        ],
    )(x, wa, wg, wu, wd, sb)

    return out.reshape(orig_shape), gs.reshape(orig_shape[:-1] + (E,))
